# Initial kernel scaffold; baseline (speedup 1.0000x reference)
import jax
import jax.numpy as jnp
from jax.experimental import pallas as pl


def kernel(rx, h):
    raise NotImplementedError("write your pallas kernel here")


# speedup vs baseline: 8.5219x; 8.5219x over previous
import numpy as np
import jax
import jax.numpy as jnp
from jax.experimental import pallas as pl
from jax.experimental.pallas import tpu as pltpu

_MEMORY_LENGTH = 8
_N_STATES = 256
_SNR = 10.0
_HALF = 0.5
_SEQ_LEN = 50000
_CHUNK = 1000


def _state_symbols():
    dec = np.arange(_N_STATES).astype(np.uint8).reshape(-1, 1)
    bits = np.unpackbits(dec, axis=1).astype(int)
    syms = (-1.0) ** bits[:, -_MEMORY_LENGTH:]
    return jnp.asarray(syms, dtype=jnp.float32)


def _body(rx_ref, sp_ref, ssum_ref, det_ref, cbits_ref, cword_ref,
          v_ref, vhist_ref, llr_ref):
    c = _CHUNK
    snr_value = 10.0 ** (_SNR / 10.0)
    sigma = snr_value ** (-_HALF)
    denom = 2.0 * sigma ** 2

    i = pl.program_id(0)

    @pl.when(i == 0)
    def _init():
        v_ref[...] = jnp.zeros((1, 128), dtype=jnp.float32)

    y = rx_ref[...]
    d = y - sp_ref[...]
    p2 = jnp.exp(-d ** 2 / denom)
    ssum = ssum_ref[...]
    probs = p2 / ssum
    priors = jnp.log(probs)
    llr_ref[...] = -priors

    pmax = jnp.max(probs, axis=1, keepdims=True)
    iota256 = jax.lax.broadcasted_iota(jnp.int32, (c, _N_STATES), 1)
    amax = jnp.min(jnp.where(probs == pmax, iota256, _N_STATES), axis=1)
    cbits_ref[...] = (amax % 2).astype(jnp.int32).reshape(c, 1)
    cword_ref[...] = pmax

    def step(t, v):
        vhist_ref[pl.ds(t, 1), :] = v
        lr = llr_ref[pl.ds(t, 1), :]
        u = jnp.concatenate([v, v], axis=1)
        s = u + lr
        return jnp.min(s.reshape(1, 128, 2), axis=2)

    v = jax.lax.fori_loop(0, c, step, v_ref[...])
    v_ref[...] = v

    vh = vhist_ref[...]
    vmin = jnp.min(vh, axis=1, keepdims=True)
    iota128 = jax.lax.broadcasted_iota(jnp.int32, (c, 128), 1)
    amin = jnp.min(jnp.where(vh == vmin, iota128, 128), axis=1)
    det_ref[...] = (amin % 2).astype(jnp.float32).reshape(c, 1)


def kernel(rx, h):
    syms = _state_symbols()
    sp = syms @ h.T
    sp_row = sp.T

    snr_value = 10.0 ** (_SNR / 10.0)
    sigma = snr_value ** (-_HALF)
    ssum = jnp.exp(-(rx - sp_row) ** 2 / (2.0 * sigma ** 2)).sum(axis=1).reshape(-1, 1)

    grid = _SEQ_LEN // _CHUNK
    out_shapes = (
        jax.ShapeDtypeStruct((_SEQ_LEN, 1), jnp.float32),
        jax.ShapeDtypeStruct((_SEQ_LEN, 1), jnp.int32),
        jax.ShapeDtypeStruct((_SEQ_LEN, 1), jnp.float32),
    )
    detected, cbits, cword = pl.pallas_call(
        _body,
        grid=(grid,),
        in_specs=[
            pl.BlockSpec((_CHUNK, 1), lambda i: (i, 0)),
            pl.BlockSpec((1, _N_STATES), lambda i: (0, 0)),
            pl.BlockSpec((_CHUNK, 1), lambda i: (i, 0)),
        ],
        out_specs=(
            pl.BlockSpec((_CHUNK, 1), lambda i: (i, 0)),
            pl.BlockSpec((_CHUNK, 1), lambda i: (i, 0)),
            pl.BlockSpec((_CHUNK, 1), lambda i: (i, 0)),
        ),
        out_shape=out_shapes,
        scratch_shapes=[
            pltpu.VMEM((1, 128), jnp.float32),
            pltpu.VMEM((_CHUNK, 128), jnp.float32),
            pltpu.VMEM((_CHUNK, _N_STATES), jnp.float32),
        ],
    )(rx, sp_row, ssum)
    return detected, cbits, cword


# speedup vs baseline: 32.1661x; 3.7745x over previous
import numpy as np
import jax
import jax.numpy as jnp
from jax.experimental import pallas as pl
from jax.experimental.pallas import tpu as pltpu

_MEMORY_LENGTH = 8
_N_STATES = 256
_SNR = 10.0
_HALF = 0.5
_SEQ_LEN = 50000
_CHUNK = 1792
_T_PAD = 50176
_REPS = _CHUNK // 7


def _rotr7(p, k):
    k = k % 7
    return ((p >> k) | (p << (7 - k))) & 127


def _make_tables():
    p = np.arange(128)
    q = np.arange(256)
    jidx7 = np.zeros((7, 256), np.int32)
    iot7 = np.zeros((7, 128), np.int32)
    for r in range(7):
        k2 = (r + 1) % 7
        jidx7[r] = (_rotr7(q & 127, k2) << 1) | (q >> 7)
        iot7[r] = _rotr7(p, r)
    return jidx7, iot7


_JIDX7, _IOT7 = _make_tables()
_JIDX = jnp.asarray(np.tile(_JIDX7, (_REPS, 1)))
_IOT = jnp.asarray(np.tile(_IOT7, (_REPS, 1)))


def _state_symbols():
    dec = np.arange(_N_STATES).astype(np.uint8).reshape(-1, 1)
    bits = np.unpackbits(dec, axis=1).astype(int)
    syms = (-1.0) ** bits[:, -_MEMORY_LENGTH:]
    return jnp.asarray(syms, dtype=jnp.float32)


def _body(rx_ref, spsel_ref, ssum_ref, jidx_ref, iot_ref,
          det_ref, cbits_ref, cword_ref,
          v_ref, vhist_ref, llr_ref):
    c = _CHUNK
    snr_value = 10.0 ** (_SNR / 10.0)
    sigma = snr_value ** (-_HALF)
    denom = 2.0 * sigma ** 2

    i = pl.program_id(0)

    @pl.when(i == 0)
    def _init():
        v_ref[...] = jnp.zeros((1, 128), dtype=jnp.float32)

    y = rx_ref[...]
    d = y - spsel_ref[...]
    p2 = jnp.exp(-d ** 2 / denom)
    probs = p2 / ssum_ref[...]
    llr_ref[...] = -jnp.log(probs)

    pmax = jnp.max(probs, axis=1, keepdims=True)
    cand = jnp.where(probs == pmax, jidx_ref[...], _N_STATES)
    cbits_ref[...] = (jnp.min(cand, axis=1) % 2).astype(jnp.int32).reshape(c, 1)
    cword_ref[...] = pmax

    lane = jax.lax.broadcasted_iota(jnp.int32, (1, 128), 1)
    masks = [(lane & (1 << r)) != 0 for r in range(7)]

    def it(mi, x):
        base = mi * 7
        for r in range(7):
            t = base + r
            vhist_ref[pl.ds(t, 1), :] = x
            row = llr_ref[pl.ds(t, 1), :]
            l0 = row[:, 0:128]
            l1 = row[:, 128:256]
            m = 1 << r
            xm = pltpu.roll(x, m, 1)
            xp = pltpu.roll(x, 128 - m, 1)
            y0 = jnp.where(masks[r], xm, x)
            y1 = jnp.where(masks[r], x, xp)
            x = jnp.minimum(y0 + l0, y1 + l1)
        return x

    x = jax.lax.fori_loop(0, _REPS, it, v_ref[...])
    v_ref[...] = x

    vh = vhist_ref[...]
    vmin = jnp.min(vh, axis=1, keepdims=True)
    candd = jnp.where(vh == vmin, iot_ref[...], 128)
    det_ref[...] = (jnp.min(candd, axis=1) % 2).astype(jnp.float32).reshape(c, 1)


def kernel(rx, h):
    syms = _state_symbols()
    sp = syms @ h.T
    sp_row = sp.T

    spsel7 = sp.reshape(-1)[_JIDX7.reshape(-1)].reshape(7, 256)
    spsel = jnp.tile(spsel7, (_REPS, 1))

    rx_pad = jnp.concatenate(
        [rx, jnp.zeros((_T_PAD - _SEQ_LEN, 1), jnp.float32)], axis=0)

    snr_value = 10.0 ** (_SNR / 10.0)
    sigma = snr_value ** (-_HALF)
    ssum = jnp.exp(-(rx_pad - sp_row) ** 2 / (2.0 * sigma ** 2)).sum(axis=1).reshape(-1, 1)

    grid = _T_PAD // _CHUNK
    out_shapes = (
        jax.ShapeDtypeStruct((_T_PAD, 1), jnp.float32),
        jax.ShapeDtypeStruct((_T_PAD, 1), jnp.int32),
        jax.ShapeDtypeStruct((_T_PAD, 1), jnp.float32),
    )
    detected, cbits, cword = pl.pallas_call(
        _body,
        grid=(grid,),
        in_specs=[
            pl.BlockSpec((_CHUNK, 1), lambda i: (i, 0)),
            pl.BlockSpec((_CHUNK, _N_STATES), lambda i: (0, 0)),
            pl.BlockSpec((_CHUNK, 1), lambda i: (i, 0)),
            pl.BlockSpec((_CHUNK, _N_STATES), lambda i: (0, 0)),
            pl.BlockSpec((_CHUNK, 128), lambda i: (0, 0)),
        ],
        out_specs=(
            pl.BlockSpec((_CHUNK, 1), lambda i: (i, 0)),
            pl.BlockSpec((_CHUNK, 1), lambda i: (i, 0)),
            pl.BlockSpec((_CHUNK, 1), lambda i: (i, 0)),
        ),
        out_shape=out_shapes,
        scratch_shapes=[
            pltpu.VMEM((1, 128), jnp.float32),
            pltpu.VMEM((_CHUNK, 128), jnp.float32),
            pltpu.VMEM((_CHUNK, _N_STATES), jnp.float32),
        ],
    )(rx_pad, spsel, ssum, _JIDX, _IOT)
    return (detected[:_SEQ_LEN], cbits[:_SEQ_LEN], cword[:_SEQ_LEN])


# speedup vs baseline: 32.1842x; 1.0006x over previous
import numpy as np
import jax
import jax.numpy as jnp
from jax.experimental import pallas as pl
from jax.experimental.pallas import tpu as pltpu

_MEMORY_LENGTH = 8
_N_STATES = 256
_SNR = 10.0
_HALF = 0.5
_SEQ_LEN = 50000
_CHUNK = 1792
_T_PAD = 50176
_REPS = _CHUNK // 7


def _rotr7(p, k):
    k = k % 7
    return ((p >> k) | (p << (7 - k))) & 127


def _make_tables():
    p = np.arange(128)
    q = np.arange(256)
    jidx7 = np.zeros((7, 256), np.int32)
    iot7 = np.zeros((7, 128), np.int32)
    for r in range(7):
        k2 = (r + 1) % 7
        jidx7[r] = (_rotr7(q & 127, k2) << 1) | (q >> 7)
        iot7[r] = _rotr7(p, r)
    return jidx7, iot7


_JIDX7, _IOT7 = _make_tables()
_JIDX = np.tile(_JIDX7, (_REPS, 1))
_IOT = np.tile(_IOT7, (_REPS, 1))


def _state_symbols():
    dec = np.arange(_N_STATES).astype(np.uint8).reshape(-1, 1)
    bits = np.unpackbits(dec, axis=1).astype(int)
    syms = (-1.0) ** bits[:, -_MEMORY_LENGTH:]
    return jnp.asarray(syms, dtype=jnp.float32)


def _body(rx_ref, spsel_ref, ssum_ref, jidx_ref, iot_ref,
          det_ref, cbits_ref, cword_ref,
          v_ref, vhist_ref, llr_ref):
    c = _CHUNK
    snr_value = 10.0 ** (_SNR / 10.0)
    sigma = snr_value ** (-_HALF)
    denom = 2.0 * sigma ** 2

    i = pl.program_id(0)

    @pl.when(i == 0)
    def _init():
        v_ref[...] = jnp.zeros((1, 128), dtype=jnp.float32)

    y = rx_ref[...]
    d = y - spsel_ref[...]
    p2 = jnp.exp(-d ** 2 / denom)
    probs = p2 / ssum_ref[...]
    llr_ref[...] = -jnp.log(probs)

    pmax = jnp.max(probs, axis=1, keepdims=True)
    cand = jnp.where(probs == pmax, jidx_ref[...], _N_STATES)
    cbits_ref[...] = (jnp.min(cand, axis=1) % 2).astype(jnp.int32).reshape(c, 1)
    cword_ref[...] = pmax

    lane = jax.lax.broadcasted_iota(jnp.int32, (1, 128), 1)
    masks = [(lane & (1 << r)) != 0 for r in range(7)]

    def it(mi, x):
        base = mi * 7
        for r in range(7):
            t = base + r
            vhist_ref[pl.ds(t, 1), :] = x
            row = llr_ref[pl.ds(t, 1), :]
            l0 = row[:, 0:128]
            l1 = row[:, 128:256]
            m = 1 << r
            xm = pltpu.roll(x, m, 1)
            xp = pltpu.roll(x, 128 - m, 1)
            y0 = jnp.where(masks[r], xm, x)
            y1 = jnp.where(masks[r], x, xp)
            x = jnp.minimum(y0 + l0, y1 + l1)
        return x

    x = jax.lax.fori_loop(0, _REPS, it, v_ref[...])
    v_ref[...] = x

    vh = vhist_ref[...]
    vmin = jnp.min(vh, axis=1, keepdims=True)
    candd = jnp.where(vh == vmin, iot_ref[...], 128)
    det_ref[...] = (jnp.min(candd, axis=1) % 2).astype(jnp.float32).reshape(c, 1)


def kernel(rx, h):
    syms = _state_symbols()
    sp = syms @ h.T
    sp_row = sp.T

    spsel7 = sp.reshape(-1)[_JIDX7.reshape(-1)].reshape(7, 256)
    spsel = jnp.tile(spsel7, (_REPS, 1))

    rx_pad = jnp.concatenate(
        [rx, jnp.zeros((_T_PAD - _SEQ_LEN, 1), jnp.float32)], axis=0)

    snr_value = 10.0 ** (_SNR / 10.0)
    sigma = snr_value ** (-_HALF)
    ssum = jnp.exp(-(rx_pad - sp_row) ** 2 / (2.0 * sigma ** 2)).sum(axis=1).reshape(-1, 1)

    grid = _T_PAD // _CHUNK
    out_shapes = (
        jax.ShapeDtypeStruct((_T_PAD, 1), jnp.float32),
        jax.ShapeDtypeStruct((_T_PAD, 1), jnp.int32),
        jax.ShapeDtypeStruct((_T_PAD, 1), jnp.float32),
    )
    detected, cbits, cword = pl.pallas_call(
        _body,
        grid=(grid,),
        in_specs=[
            pl.BlockSpec((_CHUNK, 1), lambda i: (i, 0)),
            pl.BlockSpec((_CHUNK, _N_STATES), lambda i: (0, 0)),
            pl.BlockSpec((_CHUNK, 1), lambda i: (i, 0)),
            pl.BlockSpec((_CHUNK, _N_STATES), lambda i: (0, 0)),
            pl.BlockSpec((_CHUNK, 128), lambda i: (0, 0)),
        ],
        out_specs=(
            pl.BlockSpec((_CHUNK, 1), lambda i: (i, 0)),
            pl.BlockSpec((_CHUNK, 1), lambda i: (i, 0)),
            pl.BlockSpec((_CHUNK, 1), lambda i: (i, 0)),
        ),
        out_shape=out_shapes,
        scratch_shapes=[
            pltpu.VMEM((1, 128), jnp.float32),
            pltpu.VMEM((_CHUNK, 128), jnp.float32),
            pltpu.VMEM((_CHUNK, _N_STATES), jnp.float32),
        ],
    )(rx_pad, spsel, ssum, jnp.asarray(_JIDX), jnp.asarray(_IOT))
    return (detected[:_SEQ_LEN], cbits[:_SEQ_LEN], cword[:_SEQ_LEN])
